# baseline (speedup 1.0000x reference)
import functools

import jax
import jax.numpy as jnp
from jax import lax
from jax.experimental import pallas as pl
from jax.experimental.pallas import tpu as pltpu
from jax.experimental.pallas import tpu_sc as plsc

_NUM_INTERVENTIONS = 128
_BATCH = 16384
_DIM = 512
_NC = 2
_NS = 16
_NW = _NC * _NS
_ROWS_PER_W = _BATCH // _NW
_CHUNK = 64
_NCHUNK = _ROWS_PER_W // _CHUNK

_mesh = plsc.VectorSubcoreMesh(core_axis_name="c", subcore_axis_name="s")


@functools.partial(
    pl.kernel,
    mesh=_mesh,
    out_type=jax.ShapeDtypeStruct((_BATCH * _DIM,), jnp.float32),
    compiler_params=pltpu.CompilerParams(needs_layout_passes=False),
    scratch_types=[
        pltpu.VMEM((_NUM_INTERVENTIONS,), jnp.int32),
        pltpu.VMEM((_CHUNK * _DIM,), jnp.float32),
        pltpu.VMEM((_CHUNK * _DIM,), jnp.float32),
    ],
)
def _sc_fix_columns(x_hbm, c_hbm, idx_hbm, out_hbm, idx_v, x_v, c_v):
    wid = lax.axis_index("s") * _NC + lax.axis_index("c")
    base = wid * _ROWS_PER_W * _DIM
    pltpu.sync_copy(idx_hbm, idx_v)

    def chunk_body(k, carry):
        e0 = base + k * (_CHUNK * _DIM)
        pltpu.sync_copy(x_hbm.at[pl.ds(e0, _CHUNK * _DIM)], x_v)
        pltpu.sync_copy(c_hbm.at[pl.ds(e0, _CHUNK * _DIM)], c_v)

        def row_body(r, c2):
            rbase = jnp.full((16,), r * _DIM, jnp.int32)
            for j in range(_NUM_INTERVENTIONS // 16):
                fidx = rbase + idx_v[pl.ds(j * 16, 16)]
                vals = plsc.load_gather(c_v, [fidx])
                plsc.store_scatter(x_v, [fidx], 1.0 - vals)
            return c2

        lax.fori_loop(0, _CHUNK, row_body, 0)
        pltpu.sync_copy(x_v, out_hbm.at[pl.ds(e0, _CHUNK * _DIM)])
        return carry

    lax.fori_loop(0, _NCHUNK, chunk_body, 0)


def kernel(x, concepts):
    batch, dim = x.shape
    idx = jax.random.permutation(jax.random.key(42), dim)[:_NUM_INTERVENTIONS]
    out = _sc_fix_columns(
        x.reshape(-1), concepts.reshape(-1), idx.astype(jnp.int32)
    )
    return out.reshape(batch, dim)


# speedup vs baseline: 4.6017x; 4.6017x over previous
import jax
import jax.numpy as jnp
from jax.experimental import pallas as pl

_NUM_INTERVENTIONS = 128
_ROW_BLOCK = 2048


def _masked_select_body(mask_ref, x_ref, c_ref, o_ref):
    m = mask_ref[...]
    o_ref[...] = jnp.where(m > 0.5, 1.0 - c_ref[...], x_ref[...])


def kernel(x, concepts):
    batch, dim = x.shape
    idx = jax.random.permutation(jax.random.key(42), dim)[:_NUM_INTERVENTIONS]
    mask = jnp.zeros((1, dim), jnp.float32).at[0, idx].set(1.0)

    rows = min(_ROW_BLOCK, batch)
    grid = (batch // rows,)
    return pl.pallas_call(
        _masked_select_body,
        grid=grid,
        in_specs=[
            pl.BlockSpec((1, dim), lambda i: (0, 0)),
            pl.BlockSpec((rows, dim), lambda i: (i, 0)),
            pl.BlockSpec((rows, dim), lambda i: (i, 0)),
        ],
        out_specs=pl.BlockSpec((rows, dim), lambda i: (i, 0)),
        out_shape=jax.ShapeDtypeStruct((batch, dim), x.dtype),
    )(mask, x, concepts)
